# Initial kernel scaffold; baseline (speedup 1.0000x reference)
import jax
import jax.numpy as jnp
from jax.experimental import pallas as pl


def kernel(seq, raw_text, table, W, b):
    raise NotImplementedError("write your pallas kernel here")


# speedup vs baseline: 1.9225x; 1.9225x over previous
import functools

import jax
import jax.numpy as jnp
from jax import lax
from jax.experimental import pallas as pl
from jax.experimental.pallas import tpu as pltpu
from jax.experimental.pallas import tpu_sc as plsc


def _emb_sum_sc(seq, table):
  L, B = seq.shape
  _, E = table.shape
  info = plsc.get_sparse_core_info()
  NC, NS = info.num_cores, info.num_subcores
  NW = NC * NS
  bpw = B // NW
  LC = 4
  NLC = L // LC
  NG = bpw // 128
  EV = E // 16

  mesh = plsc.VectorSubcoreMesh(core_axis_name="c", subcore_axis_name="s")

  @functools.partial(
      pl.kernel,
      mesh=mesh,
      out_type=jax.ShapeDtypeStruct((B, E), jnp.float32),
      compiler_params=pltpu.CompilerParams(use_tc_tiling_on_sc=False),
      scratch_types=[
          pltpu.VMEM((2, LC, bpw), jnp.int32),
          pltpu.VMEM((2, bpw, E), jnp.float32),
          pltpu.VMEM((bpw, E), jnp.float32),
          pltpu.SemaphoreType.DMA,
          pltpu.SemaphoreType.DMA,
          pltpu.SemaphoreType.DMA,
      ],
  )
  def k(seq_h, table_h, out_h, idx_v, rows_v, acc_v, sem_i, sem_r0, sem_r1):
    wid = lax.axis_index("s") * NC + lax.axis_index("c")
    base = wid * bpw
    sems = (sem_r0, sem_r1)

    zeros = jnp.zeros((16,), jnp.float32)
    def zbody(i0, _):
      for r in range(4):
        i = i0 * 4 + r
        for j in range(EV):
          acc_v[i, pl.ds(j * 16, 16)] = zeros
      return 0
    lax.fori_loop(0, bpw // 4, zbody, 0)

    def issue_gathers(cbuf, li, rbuf):
      for g in range(NG):
        pltpu.async_copy(
            table_h.at[idx_v.at[cbuf, li, pl.ds(g * 128, 128)]],
            rows_v.at[rbuf, pl.ds(g * 128, 128), :],
            sems[rbuf])

    def drain_rows(rbuf):
      pltpu.make_async_copy(
          table_h.at[pl.ds(0, bpw), :], rows_v.at[rbuf], sems[rbuf]).wait()

    def accum(rbuf):
      def abody(i0, _):
        for r in range(4):
          i = i0 * 4 + r
          for j in range(EV):
            sl = pl.ds(j * 16, 16)
            acc_v[i, sl] = acc_v[i, sl] + rows_v[rbuf, i, sl]
        return 0
      lax.fori_loop(0, bpw // 4, abody, 0)

    pltpu.sync_copy(seq_h.at[pl.ds(0, LC), pl.ds(base, bpw)], idx_v.at[0])
    issue_gathers(0, 0, 0)

    def chunk(c, _):
      cbuf = c % 2
      nbuf = (c + 1) % 2

      @pl.when(c < NLC - 1)
      def _():
        pltpu.async_copy(
            seq_h.at[pl.ds((c + 1) * LC, LC), pl.ds(base, bpw)],
            idx_v.at[nbuf], sem_i)

      for li in range(LC):
        rbuf = li % 2
        if li < LC - 1:
          issue_gathers(cbuf, li + 1, 1 - rbuf)
        else:
          @pl.when(c < NLC - 1)
          def _():
            pltpu.make_async_copy(
                seq_h.at[pl.ds(0, LC), pl.ds(base, bpw)],
                idx_v.at[nbuf], sem_i).wait()
            issue_gathers(nbuf, 0, 1 - rbuf)
        drain_rows(rbuf)
        accum(rbuf)
      return 0

    lax.fori_loop(0, NLC, chunk, 0)
    pltpu.sync_copy(acc_v, out_h.at[pl.ds(base, bpw), :])

  return k(seq, table)


def _linear_tc(emb, W, b):
  B, E = emb.shape
  O = W.shape[0]
  BLK = 2048

  def body(e_ref, w_ref, b_ref, o_ref):
    e = e_ref[...]
    w = w_ref[...]
    o_ref[...] = lax.dot_general(
        e, w, (((1,), (1,)), ((), ())),
        preferred_element_type=jnp.float32) + b_ref[...]

  return pl.pallas_call(
      body,
      grid=(B // BLK,),
      in_specs=[
          pl.BlockSpec((BLK, E), lambda i: (i, 0)),
          pl.BlockSpec((O, E), lambda i: (0, 0)),
          pl.BlockSpec((1, O), lambda i: (0, 0)),
      ],
      out_specs=pl.BlockSpec((BLK, O), lambda i: (i, 0)),
      out_shape=jax.ShapeDtypeStruct((B, O), jnp.float32),
  )(emb, W, b.reshape(1, O))


def kernel(seq, raw_text, table, W, b):
  del raw_text
  emb = _emb_sum_sc(seq, table)
  return _linear_tc(emb, W, b)


# speedup vs baseline: 2.1225x; 1.1040x over previous
import functools

import jax
import jax.numpy as jnp
from jax import lax
from jax.experimental import pallas as pl
from jax.experimental.pallas import tpu as pltpu
from jax.experimental.pallas import tpu_sc as plsc


def _emb_sum_sc(seq, table):
  L, B = seq.shape
  _, E = table.shape
  info = plsc.get_sparse_core_info()
  NC, NS = info.num_cores, info.num_subcores
  NW = NC * NS
  bpw = B // NW
  BC = 128
  NP = bpw // BC
  SPG = 4
  NU = L // SPG
  NPAIR = NU // 2
  CH = 40
  NCH = L // CH
  CHU = CH // SPG
  PPC = CHU // 2
  EV = E // 16

  mesh = plsc.VectorSubcoreMesh(core_axis_name="c", subcore_axis_name="s")

  @functools.partial(
      pl.kernel,
      mesh=mesh,
      out_type=jax.ShapeDtypeStruct((B, E), jnp.float32),
      compiler_params=pltpu.CompilerParams(use_tc_tiling_on_sc=False),
      scratch_types=[
          pltpu.VMEM((2, CH, BC), jnp.int32),
          pltpu.VMEM((2, SPG, BC, E), jnp.float32),
          pltpu.VMEM((BC, E), jnp.float32),
          pltpu.SemaphoreType.DMA,
          pltpu.SemaphoreType.DMA,
          pltpu.SemaphoreType.DMA,
      ],
  )
  def k(seq_h, table_h, out_h, idx_v, rows_v, acc_v, sem_i, sem_r0, sem_r1):
    wid = lax.axis_index("s") * NC + lax.axis_index("c")
    base = wid * bpw
    sems = (sem_r0, sem_r1)
    zeros = jnp.zeros((16,), jnp.float32)

    def issue_unit(u, g):
      for kk in range(SPG):
        s = u * SPG + kk
        ch = s // CH
        li = s % CH
        pltpu.async_copy(
            table_h.at[idx_v.at[ch % 2, li, :]],
            rows_v.at[g, kk, :, :],
            sems[g])

    def drain_unit(g):
      for kk in range(SPG):
        pltpu.make_async_copy(
            table_h.at[pl.ds(0, BC), :], rows_v.at[g, kk], sems[g]).wait()

    def accum_unit(g):
      def abody(i0, _):
        for r in range(4):
          i = i0 * 4 + r
          for j in range(EV):
            sl = pl.ds(j * 16, 16)
            v = acc_v[i, sl]
            for kk in range(SPG):
              v = v + rows_v[g, kk, i, sl]
            acc_v[i, sl] = v
        return 0
      lax.fori_loop(0, BC // 4, abody, 0)

    def zero_acc():
      def zbody(i0, _):
        for r in range(4):
          i = i0 * 4 + r
          for j in range(EV):
            acc_v[i, pl.ds(j * 16, 16)] = zeros
        return 0
      lax.fori_loop(0, BC // 4, zbody, 0)

    def pass_body(p, _):
      pb = base + p * BC
      zero_acc()
      pltpu.sync_copy(seq_h.at[pl.ds(0, CH), pl.ds(pb, BC)], idx_v.at[0])
      issue_unit(jnp.int32(0), 0)

      def pair(t, _):
        u0 = t * 2

        @pl.when(jnp.logical_and(t % PPC == 0, t < (NCH - 1) * PPC))
        def _():
          ch_next = t // PPC + 1
          pltpu.async_copy(
              seq_h.at[pl.ds(ch_next * CH, CH), pl.ds(pb, BC)],
              idx_v.at[ch_next % 2], sem_i)

        issue_unit(u0 + 1, 1)
        drain_unit(0)
        accum_unit(0)

        @pl.when(jnp.logical_and((u0 + 2) % CHU == 0, u0 + 2 < NU))
        def _():
          pltpu.make_async_copy(
              seq_h.at[pl.ds(0, CH), pl.ds(0, BC)],
              idx_v.at[((u0 + 2) // CHU) % 2], sem_i).wait()

        @pl.when(u0 + 2 < NU)
        def _():
          issue_unit(u0 + 2, 0)
        drain_unit(1)
        accum_unit(1)
        return 0

      lax.fori_loop(0, NPAIR, pair, 0)
      pltpu.sync_copy(acc_v, out_h.at[pl.ds(pb, BC), :])
      return 0

    lax.fori_loop(0, NP, pass_body, 0)

  return k(seq, table)


def _linear_tc(emb, W, b):
  B, E = emb.shape
  O = W.shape[0]
  BLK = 2048

  def body(e_ref, w_ref, b_ref, o_ref):
    e = e_ref[...]
    w = w_ref[...]
    o_ref[...] = lax.dot_general(
        e, w, (((1,), (1,)), ((), ())),
        preferred_element_type=jnp.float32) + b_ref[...]

  return pl.pallas_call(
      body,
      grid=(B // BLK,),
      in_specs=[
          pl.BlockSpec((BLK, E), lambda i: (i, 0)),
          pl.BlockSpec((O, E), lambda i: (0, 0)),
          pl.BlockSpec((1, O), lambda i: (0, 0)),
      ],
      out_specs=pl.BlockSpec((BLK, O), lambda i: (i, 0)),
      out_shape=jax.ShapeDtypeStruct((B, O), jnp.float32),
  )(emb, W, b.reshape(1, O))


def kernel(seq, raw_text, table, W, b):
  del raw_text
  emb = _emb_sum_sc(seq, table)
  return _linear_tc(emb, W, b)
